# baseline (speedup 1.0000x reference)
import functools

import jax
import jax.numpy as jnp
from jax import lax
from jax.experimental import pallas as pl
from jax.experimental.pallas import tpu as pltpu
from jax.experimental.pallas import tpu_sc as plsc

B = 4096
L = 50
D = 64
NC = 2
NS = 16
NW = NC * NS
RPW = B // NW
RPC = 2
CPW = RPW // RPC
IC = RPC * L
NV = D // 16
NBUF = 4

_MESH = plsc.VectorSubcoreMesh(core_axis_name="c", subcore_axis_name="s")


@functools.partial(
    pl.kernel,
    out_type=jax.ShapeDtypeStruct((B, D), jnp.float32),
    mesh=_MESH,
    compiler_params=pltpu.CompilerParams(use_tc_tiling_on_sc=False),
    scratch_types=[
        pltpu.VMEM((CPW, IC), jnp.int32),
        pltpu.VMEM((NBUF, IC, D), jnp.float32),
        pltpu.VMEM((RPW, D), jnp.float32),
        pltpu.SemaphoreType.DMA,
        pltpu.SemaphoreType.DMA,
        pltpu.SemaphoreType.DMA,
        pltpu.SemaphoreType.DMA,
    ],
)
def _embed_sum(idx_hbm, table_hbm, out_hbm, idx_v, rows_v, out_v,
               sem0, sem1, sem2, sem3):
    wid = lax.axis_index("s") * NC + lax.axis_index("c")
    pltpu.sync_copy(idx_hbm.at[wid], idx_v)
    sems = (sem0, sem1, sem2, sem3)

    def issue(ch, buf):
        pltpu.async_copy(table_hbm.at[idx_v.at[ch]], rows_v.at[buf], sems[buf])

    def wait(ch, buf):
        pltpu.make_async_copy(
            table_hbm.at[idx_v.at[ch]], rows_v.at[buf], sems[buf]).wait()

    def process(ch, buf):
        for rr in range(RPC):
            zero = jnp.zeros((16,), jnp.float32)

            @plsc.parallel_loop(0, L, unroll=10, carry=(zero,) * NV)
            def acc(j, accs):
                return tuple(
                    accs[d] + rows_v[buf, rr * L + j, pl.ds(d * 16, 16)]
                    for d in range(NV))

            row = ch * RPC + rr
            for d in range(NV):
                out_v[row, pl.ds(d * 16, 16)] = acc[d]

    for b in range(NBUF - 1):
        issue(b, b)

    def body(i, carry):
        for b in range(NBUF):
            ch = NBUF * i + b

            @pl.when(ch + NBUF - 1 < CPW)
            def _():
                issue(ch + NBUF - 1, (b - 1) % NBUF)

            wait(ch, b)
            process(ch, b)
        return carry

    lax.fori_loop(0, CPW // NBUF, body, 0)
    pltpu.sync_copy(out_v, out_hbm.at[pl.ds(wid * RPW, RPW)])


V = 100000
WC = 128
NFULL = V // WC
TAILC = V - NFULL * WC
JPW = (NFULL + NW - 1) // NW


@functools.partial(
    pl.kernel,
    out_type=jax.ShapeDtypeStruct((V * D,), jnp.float32),
    mesh=_MESH,
    compiler_params=pltpu.CompilerParams(needs_layout_passes=False),
    scratch_types=[
        pltpu.VMEM((2, D, WC), jnp.float32),
        pltpu.VMEM((WC * D,), jnp.float32),
        pltpu.VMEM((WC * D,), jnp.float32),
        pltpu.SemaphoreType.DMA,
        pltpu.SemaphoreType.DMA,
        pltpu.SemaphoreType.DMA,
        pltpu.SemaphoreType.DMA,
    ],
)
def _transpose(tableT_hbm, tail_hbm, out_hbm, stage_v, flat0_v, flat1_v,
               si0, si1, so0, so1):
    wid = lax.axis_index("s") * NC + lax.axis_index("c")
    sin = (si0, si1)
    sout = (so0, so1)
    flats = (flat0_v, flat1_v)
    lane = lax.iota(jnp.int32, 16)
    dvecs = [lane + (16 * k) for k in range(NV)]
    rotv = [(lane + r) & 15 for r in range(16)]
    sv = [rotv[r] * D + lane for r in range(16)]

    def chunk_of(j):
        return j * NW + wid

    def issue_in(j, b):
        t = chunk_of(j)

        @pl.when(t < NFULL)
        def _():
            pltpu.async_copy(tableT_hbm.at[:, pl.ds(t * WC, WC)],
                             stage_v.at[b], sin[b])

    def wait_in(j, b):
        t = chunk_of(j)

        @pl.when(t < NFULL)
        def _():
            pltpu.make_async_copy(tableT_hbm.at[:, pl.ds(t * WC, WC)],
                                  stage_v.at[b], sin[b]).wait()

    def issue_out(j, b):
        t = chunk_of(j)

        @pl.when(t < NFULL)
        def _():
            pltpu.async_copy(flats[b],
                             out_hbm.at[pl.ds(t * WC * D, WC * D)], sout[b])

    def wait_out(j, b):
        t = chunk_of(j)

        @pl.when(jnp.logical_and(t >= 0, t < NFULL))
        def _():
            pltpu.make_async_copy(flats[b],
                                  out_hbm.at[pl.ds(t * WC * D, WC * D)],
                                  sout[b]).wait()

    def process(j, b):
        t = chunk_of(j)

        @pl.when(t < NFULL)
        def _():
            @plsc.parallel_loop(0, WC // 16, unroll=2)
            def _(g):
                c0 = g * 16
                for k in range(NV):
                    base = c0 * D + 16 * k
                    for r in range(16):
                        v = plsc.load_gather(stage_v.at[b],
                                             [dvecs[k], rotv[r] + c0])
                        plsc.store_scatter(flats[b], [sv[r] + base], v)

    issue_in(0, 0)

    def body(i, carry):
        for b in range(2):
            j = 2 * i + b
            issue_in(j + 1, 1 - b)
            wait_in(j, b)
            wait_out(j - 2, b)
            process(j, b)
            issue_out(j, b)
        return carry

    lax.fori_loop(0, JPW // 2, body, 0)
    wait_in(JPW - 1, 0)
    wait_out(JPW - 3, 0)
    process(JPW - 1, 0)
    issue_out(JPW - 1, 0)
    wait_out(JPW - 2, 1)
    wait_out(JPW - 1, 0)

    @pl.when(wid == NW - 1)
    def _():
        pltpu.sync_copy(tail_hbm, flat0_v.at[pl.ds(0, TAILC * D)])
        pltpu.sync_copy(flat0_v.at[pl.ds(0, TAILC * D)],
                        out_hbm.at[pl.ds(NFULL * WC * D, TAILC * D)])


def _normalize_body(x_ref, o_ref):
    x = x_ref[...]
    ss = jnp.sum(x * x, axis=1, keepdims=True)
    o_ref[...] = x * lax.rsqrt(jnp.maximum(ss, jnp.float32(1e-24)))


_normalize = pl.pallas_call(
    _normalize_body,
    out_shape=jax.ShapeDtypeStruct((B, D), jnp.float32),
)


def kernel(text_idxs, text_len, embedding_table):
    del text_len
    idx3 = text_idxs.astype(jnp.int32).reshape(NW, CPW, IC)
    tail = embedding_table[V - TAILC:].reshape(TAILC * D)
    table_lin = _transpose(embedding_table.T, tail).reshape(V, D)
    sums = _embed_sum(idx3, table_lin)
    return _normalize(sums)


# speedup vs baseline: 1.5312x; 1.5312x over previous
import functools

import jax
import jax.numpy as jnp
from jax import lax
from jax.experimental import pallas as pl
from jax.experimental.pallas import tpu as pltpu
from jax.experimental.pallas import tpu_sc as plsc

B = 4096
L = 50
D = 64
NC = 2
NS = 16
NW = NC * NS
RPW = B // NW
RPC = 2
CPW = RPW // RPC
IC = RPC * L
NV = D // 16
NBUF = 4

_MESH = plsc.VectorSubcoreMesh(core_axis_name="c", subcore_axis_name="s")


@functools.partial(
    pl.kernel,
    out_type=jax.ShapeDtypeStruct((B, D), jnp.float32),
    mesh=_MESH,
    compiler_params=pltpu.CompilerParams(use_tc_tiling_on_sc=False),
    scratch_types=[
        pltpu.VMEM((CPW, IC), jnp.int32),
        pltpu.VMEM((NBUF, IC, D), jnp.float32),
        pltpu.VMEM((RPW, D), jnp.float32),
        pltpu.SemaphoreType.DMA,
        pltpu.SemaphoreType.DMA,
        pltpu.SemaphoreType.DMA,
        pltpu.SemaphoreType.DMA,
    ],
)
def _embed_sum(idx_hbm, table_hbm, out_hbm, idx_v, rows_v, out_v,
               sem0, sem1, sem2, sem3):
    wid = lax.axis_index("s") * NC + lax.axis_index("c")
    pltpu.sync_copy(idx_hbm.at[wid], idx_v)
    sems = (sem0, sem1, sem2, sem3)

    def issue(ch, buf):
        pltpu.async_copy(table_hbm.at[idx_v.at[ch]], rows_v.at[buf], sems[buf])

    def wait(ch, buf):
        pltpu.make_async_copy(
            table_hbm.at[idx_v.at[ch]], rows_v.at[buf], sems[buf]).wait()

    def process(ch, buf):
        for rr in range(RPC):
            zero = jnp.zeros((16,), jnp.float32)

            @plsc.parallel_loop(0, L, unroll=10, carry=(zero,) * NV)
            def acc(j, accs):
                return tuple(
                    accs[d] + rows_v[buf, rr * L + j, pl.ds(d * 16, 16)]
                    for d in range(NV))

            row = ch * RPC + rr
            for d in range(NV):
                out_v[row, pl.ds(d * 16, 16)] = acc[d]

    for b in range(NBUF - 1):
        issue(b, b)

    def body(i, carry):
        for b in range(NBUF):
            ch = NBUF * i + b

            @pl.when(ch + NBUF - 1 < CPW)
            def _():
                issue(ch + NBUF - 1, (b - 1) % NBUF)

            wait(ch, b)
            process(ch, b)
        return carry

    lax.fori_loop(0, CPW // NBUF, body, 0)
    pltpu.sync_copy(out_v, out_hbm.at[pl.ds(wid * RPW, RPW)])


V = 100000
WC = 128
NFULL = V // WC
TAILC = V - NFULL * WC
JPW = (NFULL + NW - 1) // NW


@functools.partial(
    pl.kernel,
    out_type=jax.ShapeDtypeStruct((V * D,), jnp.float32),
    mesh=_MESH,
    compiler_params=pltpu.CompilerParams(needs_layout_passes=False),
    scratch_types=[
        pltpu.VMEM((2, D, WC), jnp.float32),
        pltpu.VMEM((WC * D,), jnp.float32),
        pltpu.VMEM((WC * D,), jnp.float32),
        pltpu.SemaphoreType.DMA,
        pltpu.SemaphoreType.DMA,
        pltpu.SemaphoreType.DMA,
        pltpu.SemaphoreType.DMA,
    ],
)
def _transpose(tableT_hbm, tail_hbm, out_hbm, stage_v, flat0_v, flat1_v,
               si0, si1, so0, so1):
    wid = lax.axis_index("s") * NC + lax.axis_index("c")
    sin = (si0, si1)
    sout = (so0, so1)
    flats = (flat0_v, flat1_v)
    lane = lax.iota(jnp.int32, 16)
    dvecs = [lane + (16 * k) for k in range(NV)]
    rotv = [(lane + r) & 15 for r in range(16)]
    sv = [rotv[r] * D + lane for r in range(16)]

    def chunk_of(j):
        return j * NW + wid

    def issue_in(j, b):
        t = chunk_of(j)

        @pl.when(t < NFULL)
        def _():
            pltpu.async_copy(tableT_hbm.at[:, pl.ds(t * WC, WC)],
                             stage_v.at[b], sin[b])

    def wait_in(j, b):
        t = chunk_of(j)

        @pl.when(t < NFULL)
        def _():
            pltpu.make_async_copy(tableT_hbm.at[:, pl.ds(t * WC, WC)],
                                  stage_v.at[b], sin[b]).wait()

    def issue_out(j, b):
        t = chunk_of(j)

        @pl.when(t < NFULL)
        def _():
            pltpu.async_copy(flats[b],
                             out_hbm.at[pl.ds(t * WC * D, WC * D)], sout[b])

    def wait_out(j, b):
        t = chunk_of(j)

        @pl.when(jnp.logical_and(t >= 0, t < NFULL))
        def _():
            pltpu.make_async_copy(flats[b],
                                  out_hbm.at[pl.ds(t * WC * D, WC * D)],
                                  sout[b]).wait()

    def process(j, b):
        t = chunk_of(j)

        @pl.when(t < NFULL)
        def _():
            @plsc.parallel_loop(0, WC // 16, unroll=1)
            def _(g):
                c0 = g * 16
                for k in range(NV):
                    base = c0 * D + 16 * k
                    for r in range(16):
                        v = plsc.load_gather(stage_v.at[b],
                                             [dvecs[k], rotv[r] + c0])
                        plsc.store_scatter(flats[b], [sv[r] + base], v)

    issue_in(0, 0)

    def body(i, carry):
        for b in range(2):
            j = 2 * i + b
            issue_in(j + 1, 1 - b)
            wait_in(j, b)
            wait_out(j - 2, b)
            process(j, b)
            issue_out(j, b)
        return carry

    lax.fori_loop(0, JPW // 2, body, 0)
    wait_in(JPW - 1, 0)
    wait_out(JPW - 3, 0)
    process(JPW - 1, 0)
    issue_out(JPW - 1, 0)
    wait_out(JPW - 2, 1)
    wait_out(JPW - 1, 0)

    @pl.when(wid == NW - 1)
    def _():
        pltpu.sync_copy(tail_hbm, flat0_v.at[pl.ds(0, TAILC * D)])
        pltpu.sync_copy(flat0_v.at[pl.ds(0, TAILC * D)],
                        out_hbm.at[pl.ds(NFULL * WC * D, TAILC * D)])


def _normalize_body(x_ref, o_ref):
    x = x_ref[...]
    ss = jnp.sum(x * x, axis=1, keepdims=True)
    o_ref[...] = x * lax.rsqrt(jnp.maximum(ss, jnp.float32(1e-24)))


_normalize = pl.pallas_call(
    _normalize_body,
    out_shape=jax.ShapeDtypeStruct((B, D), jnp.float32),
)


def kernel(text_idxs, text_len, embedding_table):
    del text_len
    idx3 = text_idxs.astype(jnp.int32).reshape(NW, CPW, IC)
    tail = embedding_table[V - TAILC:].reshape(TAILC * D)
    table_lin = _transpose(embedding_table.T, tail).reshape(V, D)
    sums = _embed_sum(idx3, table_lin)
    return _normalize(sums)
